# baseline (speedup 1.0000x reference)
import functools

import jax
import jax.numpy as jnp
from jax import lax
from jax.experimental import pallas as pl
from jax.experimental.pallas import tpu as pltpu
from jax.experimental.pallas import tpu_sc as plsc

D_MODEL = 1024
N_PIX = 4096
CHUNK = 64

_info = plsc.get_sparse_core_info()
_NC, _NS = _info.num_cores, _info.num_subcores
_NW = _NC * _NS
_B_PER_W = N_PIX // _NW
_N_CHUNKS = _B_PER_W // CHUNK


def _gather_body(pix_hbm, cond_hbm, ptok_hbm, ctok_hbm, out_hbm,
                 idx_v, rows_v, cidx_v, crow_v, sem):
    wid = lax.axis_index("s") * _NC + lax.axis_index("c")
    base = wid * _B_PER_W

    pltpu.sync_copy(ptok_hbm.at[pl.ds(base, _B_PER_W)], idx_v)

    for c in range(_N_CHUNKS):
        pltpu.async_copy(
            pix_hbm.at[idx_v.at[pl.ds(c * CHUNK, CHUNK)]], rows_v, sem
        ).wait()
        pltpu.sync_copy(
            rows_v, out_hbm.at[pl.ds(1 + base + c * CHUNK, CHUNK)]
        )

    @pl.when(wid == 0)
    def _():
        pltpu.sync_copy(ctok_hbm, cidx_v)
        pltpu.async_copy(cond_hbm.at[cidx_v], crow_v, sem).wait()
        pltpu.sync_copy(crow_v.at[pl.ds(0, 1)], out_hbm.at[pl.ds(0, 1)])


@jax.jit
def _pixel_encoding(tokens, pixel_embed, cond_embed):
    mesh = plsc.VectorSubcoreMesh(core_axis_name="c", subcore_axis_name="s")
    run = functools.partial(
        pl.kernel,
        mesh=mesh,
        out_type=jax.ShapeDtypeStruct((N_PIX + 1, D_MODEL), jnp.float32),
        scratch_types=[
            pltpu.VMEM((_B_PER_W,), jnp.int32),
            pltpu.VMEM((CHUNK, D_MODEL), jnp.float32),
            pltpu.VMEM((8,), jnp.int32),
            pltpu.VMEM((8, D_MODEL), jnp.float32),
            pltpu.SemaphoreType.DMA,
        ],
        compiler_params=pltpu.CompilerParams(use_tc_tiling_on_sc=False),
    )(_gather_body)
    ptok = lax.slice(tokens, (1,), (N_PIX + 1,))
    ctok = lax.slice(tokens, (0,), (8,))
    return run(pixel_embed, cond_embed, ptok, ctok)


def kernel(tokens, pixel_embed, cond_embed):
    return _pixel_encoding(tokens, pixel_embed, cond_embed)


# speedup vs baseline: 1.0143x; 1.0143x over previous
import functools

import jax
import jax.numpy as jnp
from jax import lax
from jax.experimental import pallas as pl
from jax.experimental.pallas import tpu as pltpu
from jax.experimental.pallas import tpu_sc as plsc

D_MODEL = 1024
N_PIX = 4096
CHUNK = 32
NBUF = 3

_info = plsc.get_sparse_core_info()
_NC, _NS = _info.num_cores, _info.num_subcores
_NW = _NC * _NS
_B_PER_W = N_PIX // _NW
_N_CHUNKS = _B_PER_W // CHUNK


def _gather_body(pix_hbm, cond_hbm, ptok_hbm, ctok_hbm, out_hbm,
                 idx_v, bufs, cidx_v, crow_v, g0, g1, g2, w0, w1, w2, csem):
    gsems = [g0, g1, g2]
    wsems = [w0, w1, w2]
    wid = lax.axis_index("s") * _NC + lax.axis_index("c")
    base = wid * _B_PER_W

    pltpu.sync_copy(ptok_hbm.at[pl.ds(base, _B_PER_W)], idx_v)

    @pl.when(wid == 0)
    def _():
        pltpu.sync_copy(ctok_hbm, cidx_v)
        pltpu.async_copy(cond_hbm.at[cidx_v], crow_v, csem)

    def gather(c, b):
        return pltpu.make_async_copy(
            pix_hbm.at[idx_v.at[pl.ds(c * CHUNK, CHUNK)]], bufs.at[b],
            gsems[b],
        )

    def write(c, b):
        return pltpu.make_async_copy(
            bufs.at[b], out_hbm.at[pl.ds(1 + base + c * CHUNK, CHUNK)],
            wsems[b],
        )

    for c in range(min(NBUF, _N_CHUNKS)):
        gather(c, c).start()

    pending = {}
    for c in range(_N_CHUNKS):
        b = c % NBUF
        gather(c, b).wait()
        write(c, b).start()
        pending[b] = c
        if c + NBUF < _N_CHUNKS:
            write(c, b).wait()
            del pending[b]
            gather(c + NBUF, b).start()

    for b, c in pending.items():
        write(c, b).wait()

    @pl.when(wid == 0)
    def _():
        pltpu.make_async_copy(cond_hbm.at[cidx_v], crow_v, csem).wait()
        pltpu.sync_copy(crow_v.at[pl.ds(0, 1)], out_hbm.at[pl.ds(0, 1)])


@jax.jit
def _pixel_encoding(tokens, pixel_embed, cond_embed):
    mesh = plsc.VectorSubcoreMesh(core_axis_name="c", subcore_axis_name="s")
    run = functools.partial(
        pl.kernel,
        mesh=mesh,
        out_type=jax.ShapeDtypeStruct((N_PIX + 1, D_MODEL), jnp.float32),
        scratch_types=[
            pltpu.VMEM((_B_PER_W,), jnp.int32),
            pltpu.VMEM((NBUF, CHUNK, D_MODEL), jnp.float32),
            pltpu.VMEM((8,), jnp.int32),
            pltpu.VMEM((8, D_MODEL), jnp.float32),
        ] + [pltpu.SemaphoreType.DMA] * 7,
        compiler_params=pltpu.CompilerParams(use_tc_tiling_on_sc=False),
    )(_gather_body)
    ptok = lax.slice(tokens, (1,), (N_PIX + 1,))
    ctok = lax.slice(tokens, (0,), (8,))
    return run(pixel_embed, cond_embed, ptok, ctok)


def kernel(tokens, pixel_embed, cond_embed):
    return _pixel_encoding(tokens, pixel_embed, cond_embed)


# speedup vs baseline: 2.1713x; 2.1406x over previous
import functools

import jax
import jax.numpy as jnp
from jax import lax
from jax.experimental import pallas as pl
from jax.experimental.pallas import tpu as pltpu
from jax.experimental.pallas import tpu_sc as plsc

D_MODEL = 1024
SEQ = 4097
MAIN_ROWS = 4064
SPILL = 40
CHUNK = 32
NBUF = 3

_info = plsc.get_sparse_core_info()
_NC, _NS = _info.num_cores, _info.num_subcores
_NW = _NC * _NS
_B_PER_W = 128


def _gather_body(pix_hbm, cond_hbm, tok_hbm, tail_hbm, out_hbm, spill_hbm,
                 idx_v, tidx_v, cidx_v, crow_v, bufs,
                 g0, g1, g2, w0, w1, w2, csem):
    gsems = [g0, g1, g2]
    wsems = [w0, w1, w2]
    wid = lax.axis_index("s") * _NC + lax.axis_index("c")
    base = pl.multiple_of(wid * _B_PER_W, _B_PER_W)

    pltpu.sync_copy(tok_hbm.at[pl.ds(base, _B_PER_W)], idx_v)

    def run_ring(chunks, fix_row0):
        def gath(i, b):
            ref, off, size, _, _ = chunks[i]
            return pltpu.make_async_copy(
                pix_hbm.at[ref.at[pl.ds(off, size)]],
                bufs.at[b].at[pl.ds(0, size)], gsems[b],
            )

        def wr(i, b):
            _, _, size, dst, doff = chunks[i]
            return pltpu.make_async_copy(
                bufs.at[b].at[pl.ds(0, size)],
                dst.at[pl.ds(doff, size)], wsems[b],
            )

        for i in range(min(NBUF, len(chunks))):
            gath(i, i).start()

        pending = {}
        for i in range(len(chunks)):
            b = i % NBUF
            gath(i, b).wait()
            if fix_row0 and i == 0:
                @pl.when(wid == 0)
                def _():
                    pltpu.sync_copy(tok_hbm.at[pl.ds(0, 8)], cidx_v)
                    pltpu.async_copy(
                        cond_hbm.at[cidx_v], crow_v, csem
                    ).wait()
                    buf0 = bufs.at[b]
                    for j in range(D_MODEL // 16):
                        buf0[0, pl.ds(j * 16, 16)] = (
                            crow_v[0, pl.ds(j * 16, 16)]
                        )
            wr(i, b).start()
            pending[b] = i
            if i + NBUF < len(chunks):
                wr(i, b).wait()
                del pending[b]
                gath(i + NBUF, b).start()

        for b, i in pending.items():
            wr(i, b).wait()

    main = [(idx_v, c * CHUNK, CHUNK, out_hbm, base + c * CHUNK)
            for c in range(4)]

    @pl.when(wid < _NW - 1)
    def _():
        run_ring(main, fix_row0=True)

    @pl.when(wid == _NW - 1)
    def _():
        pltpu.sync_copy(tail_hbm, tidx_v)
        run_ring(main[:3] + [(tidx_v, 0, CHUNK, spill_hbm, 0),
                             (tidx_v, CHUNK, SPILL - CHUNK, spill_hbm, CHUNK)],
                 fix_row0=False)


def _patch_body(main_ref, spill_ref, out_ref):
    del main_ref
    out_ref[...] = spill_ref[...]


@jax.jit
def _pixel_encoding(tokens, pixel_embed, cond_embed):
    mesh = plsc.VectorSubcoreMesh(core_axis_name="c", subcore_axis_name="s")
    run = functools.partial(
        pl.kernel,
        mesh=mesh,
        out_type=(
            jax.ShapeDtypeStruct((SEQ, D_MODEL), jnp.float32),
            jax.ShapeDtypeStruct((SPILL, D_MODEL), jnp.float32),
        ),
        scratch_types=[
            pltpu.VMEM((_B_PER_W,), jnp.int32),
            pltpu.VMEM((48,), jnp.int32),
            pltpu.VMEM((8,), jnp.int32),
            pltpu.VMEM((8, D_MODEL), jnp.float32),
            pltpu.VMEM((NBUF, CHUNK, D_MODEL), jnp.float32),
        ] + [pltpu.SemaphoreType.DMA] * 7,
    )(_gather_body)
    tail = jnp.pad(lax.slice(tokens, (MAIN_ROWS,), (SEQ,)), (0, 15))
    main, spill = run(pixel_embed, cond_embed, tokens, tail)

    return pl.pallas_call(
        _patch_body,
        out_shape=jax.ShapeDtypeStruct((SEQ, D_MODEL), jnp.float32),
        grid=(SPILL // 8,),
        in_specs=[
            pl.BlockSpec(memory_space=pl.ANY),
            pl.BlockSpec((8, D_MODEL), lambda i: (i, 0)),
        ],
        out_specs=pl.BlockSpec((8, D_MODEL), lambda i: (MAIN_ROWS // 8 + i, 0)),
        input_output_aliases={0: 0},
    )(main, spill)


def kernel(tokens, pixel_embed, cond_embed):
    return _pixel_encoding(tokens, pixel_embed, cond_embed)
